# Initial kernel scaffold; baseline (speedup 1.0000x reference)
import jax
import jax.numpy as jnp
from jax.experimental import pallas as pl


def kernel(token_ids, table):
    raise NotImplementedError("write your pallas kernel here")


# speedup vs baseline: 1.3737x; 1.3737x over previous
import functools

import jax
import jax.numpy as jnp
from jax import lax
from jax.experimental import pallas as pl
from jax.experimental.pallas import tpu as pltpu
from jax.experimental.pallas import tpu_sc as plsc

VOCAB = 262144
EMBED = 1152

_info = plsc.get_sparse_core_info()
_NC, _NS = _info.num_cores, _info.num_subcores
_NW = _NC * _NS

_B = 4096
_BPW = _B // _NW
_CHUNK = 32
_NCHUNK = _BPW // _CHUNK


def _make_kernel():
    mesh = plsc.VectorSubcoreMesh(core_axis_name="c", subcore_axis_name="s")

    @functools.partial(
        pl.kernel,
        mesh=mesh,
        out_type=jax.ShapeDtypeStruct((_B, EMBED), jnp.float32),
        scratch_types=[
            pltpu.VMEM((_NCHUNK, _CHUNK), jnp.int32),
            pltpu.VMEM((2, _CHUNK, EMBED), jnp.float32),
            pltpu.SemaphoreType.DMA,
            pltpu.SemaphoreType.DMA,
            pltpu.SemaphoreType.DMA,
            pltpu.SemaphoreType.DMA,
        ],
    )
    def emb_kernel(ids_hbm, table_hbm, out_hbm, idx_v, rows_v,
                   gsem0, gsem1, osem0, osem1):
        wid = lax.axis_index("s") * _NC + lax.axis_index("c")
        base = wid * _BPW
        pltpu.sync_copy(ids_hbm.at[wid], idx_v)
        gsems = (gsem0, gsem1)
        osems = (osem0, osem1)
        g0 = pltpu.async_copy(table_hbm.at[idx_v.at[0]], rows_v.at[0], gsems[0])
        copies = [g0]
        out_copies = [None, None]
        for c in range(_NCHUNK):
            b = c % 2
            copies[c].wait()
            if c + 1 < _NCHUNK:
                if out_copies[(c + 1) % 2] is not None:
                    out_copies[(c + 1) % 2].wait()
                copies.append(
                    pltpu.async_copy(
                        table_hbm.at[idx_v.at[c + 1]],
                        rows_v.at[(c + 1) % 2],
                        gsems[(c + 1) % 2],
                    )
                )
            oc = pltpu.async_copy(
                rows_v.at[b],
                out_hbm.at[pl.ds(base + c * _CHUNK, _CHUNK)],
                osems[b],
            )
            out_copies[b] = oc
        for oc in out_copies:
            if oc is not None:
                oc.wait()

    return emb_kernel


_emb = _make_kernel()


@jax.jit
def kernel(token_ids, table):
    ids = token_ids.reshape(_NW, _NCHUNK, _CHUNK).astype(jnp.int32)
    out = _emb(ids, table)
    return out.reshape(token_ids.shape[0], token_ids.shape[1], EMBED)
